# baseline (speedup 1.0000x reference)
import functools

import jax
import jax.numpy as jnp
from jax import lax
from jax.experimental import pallas as pl
from jax.experimental.pallas import tpu as pltpu
from jax.experimental.pallas import tpu_sc as plsc

_B, _CIN, _CENC, _T = 8, 128, 256, 32768
_FLAT = _CENC * _T
_K = 512
_CH = 128
_NCHROW = _FLAT // _CH
_TCH = _T // _CH
_TBLK = 8192
_NT = _T // _TBLK

_WPR = 4
_L1N = _NCHROW // _WPR
_CANDCAP = 640
_PAD = 128


def _dense_pass_body(x_ref, wm_ref, bm_ref,
                     mask_ref, zeros_ref, cmax_ref, m_out, z_out,
                     m_run, s_run):
    t = pl.program_id(1)
    xt = x_ref[0]
    mask = jnp.dot(wm_ref[...], xt, preferred_element_type=jnp.float32)
    mask = mask + bm_ref[...]
    mask_ref[0] = mask
    zeros_ref[0] = jnp.zeros_like(mask)
    cm = jnp.max(mask.reshape(_CENC, _TBLK // _CH, _CH), axis=-1)
    cmax_ref[0] = cm.T
    tm = jnp.max(cm)
    ts = jnp.sum(jnp.exp(mask - tm))

    @pl.when(t == 0)
    def _():
        m_run[0] = tm
        s_run[0] = ts

    @pl.when(t > 0)
    def _():
        m_old = m_run[0]
        s_old = s_run[0]
        m_new = jnp.maximum(m_old, tm)
        s_run[0] = s_old * jnp.exp(m_old - m_new) + ts * jnp.exp(tm - m_new)
        m_run[0] = m_new

    @pl.when(t == _NT - 1)
    def _():
        m_out[0, 0, :] = jnp.full((128,), m_run[0], jnp.float32)
        z_out[0, 0, :] = jnp.full((128,), s_run[0], jnp.float32)


def _dense_pass(x, W_mask, b_mask):
    grid = (_B, _NT)
    out_shapes = (
        jax.ShapeDtypeStruct((_B, _CENC, _T), jnp.float32),
        jax.ShapeDtypeStruct((_B, _CENC, _T), jnp.float32),
        jax.ShapeDtypeStruct((_B, _TCH, _CENC), jnp.float32),
        jax.ShapeDtypeStruct((_B, 1, 128), jnp.float32),
        jax.ShapeDtypeStruct((_B, 1, 128), jnp.float32),
    )
    return pl.pallas_call(
        _dense_pass_body,
        grid=grid,
        in_specs=[
            pl.BlockSpec((1, _CIN, _TBLK), lambda b, t: (b, 0, t)),
            pl.BlockSpec((_CENC, _CIN), lambda b, t: (0, 0)),
            pl.BlockSpec((_CENC, 1), lambda b, t: (0, 0)),
        ],
        out_specs=(
            pl.BlockSpec((1, _CENC, _TBLK), lambda b, t: (b, 0, t)),
            pl.BlockSpec((1, _CENC, _TBLK), lambda b, t: (b, 0, t)),
            pl.BlockSpec((1, _TBLK // _CH, _CENC), lambda b, t: (b, t, 0)),
            pl.BlockSpec((1, 1, 128), lambda b, t: (b, 0, 0)),
            pl.BlockSpec((1, 1, 128), lambda b, t: (b, 0, 0)),
        ),
        out_shape=out_shapes,
        scratch_shapes=[
            pltpu.SMEM((1,), jnp.float32),
            pltpu.SMEM((1,), jnp.float32),
        ],
        compiler_params=pltpu.CompilerParams(
            dimension_semantics=("arbitrary", "arbitrary"),
        ),
    )(x, W_mask, b_mask.reshape(_CENC, 1))


_U32 = jnp.uint32
_I32 = jnp.int32


def _iota16():
    return jax.lax.iota(_I32, 16)


def _mono_key(v):
    kv = lax.bitcast_convert_type(v, _U32)
    sgn = kv >> _U32(31)
    m = jnp.where(sgn == _U32(1), _U32(0xFFFFFFFF), _U32(0x80000000))
    return kv ^ m


def _scalar(x16):
    return jnp.max(x16)


def _select_kth(get_keys, n_groups, k, hist, merged, tmp, shared, s):
    prefix = _U32(0)
    count_above = _I32(0)
    gbase = (s // 4) * 4
    for r in range(4):
        shift = 24 - 8 * r

        def zero_body(i, _):
            hist[pl.ds(i * 16, 16)] = jnp.zeros((16,), _I32)
            return 0
        lax.fori_loop(0, 256, zero_body, 0)

        pfx = prefix
        lanes = _iota16()

        def fill_body(g, _):
            key = get_keys(g)
            if r == 0:
                act = jnp.ones((16,), jnp.bool_)
            else:
                act = (key >> _U32(shift + 8)) == pfx
            digit = ((key >> _U32(shift)) & _U32(0xFF)).astype(_I32)
            addr = lanes * 256 + digit
            plsc.addupdate_scatter(hist, [addr],
                                   jnp.where(act, 1, 0).astype(_I32))
            return 0
        lax.fori_loop(0, n_groups, fill_body, 0)

        def merge_body(gg, _):
            acc = hist[pl.ds(gg * 16, 16)]
            for l in range(1, 16):
                acc = acc + hist[pl.ds(l * 256 + gg * 16, 16)]
            merged[pl.ds(gg * 16, 16)] = acc
            return 0
        lax.fori_loop(0, 16, merge_body, 0)

        pltpu.sync_copy(merged, shared.at[s])
        plsc.subcore_barrier()
        for j in range(_WPR):
            pltpu.sync_copy(shared.at[gbase + j], tmp)

            def add_body(gg, _):
                if j == 0:
                    merged[pl.ds(gg * 16, 16)] = tmp[pl.ds(gg * 16, 16)]
                else:
                    merged[pl.ds(gg * 16, 16)] = (
                        merged[pl.ds(gg * 16, 16)] + tmp[pl.ds(gg * 16, 16)])
                return 0
            lax.fori_loop(0, 16, add_body, 0)
        plsc.subcore_barrier()

        ca = count_above

        def scan_body(i, carry):
            acc, found, dstar, sfx_at, hist_at = carry
            g = 15 - i
            chunk = merged[pl.ds(g * 16, 16)]
            sfx = lax.rev(plsc.cumsum(lax.rev(chunk, (0,))), (0,))
            tot = sfx + acc
            cond = (ca + tot) >= k
            npos = _scalar(plsc.all_reduce_population_count(cond))
            has = npos > 0
            condi = jnp.where(cond, 1, 0).astype(_I32)
            ffs_rev = _scalar(plsc.all_reduce_ffs(lax.rev(condi, (0,)) == 1))
            dl = 15 - ffs_rev
            sel = lanes == dl
            z16 = jnp.zeros((16,), _I32)
            sfx_here = _scalar(jnp.where(sel, tot, z16))
            hist_here = _scalar(jnp.where(sel, chunk, z16))
            take = jnp.logical_and(has, found == 0)
            dstar = jnp.where(take, g * 16 + dl, dstar)
            sfx_at = jnp.where(take, sfx_here, sfx_at)
            hist_at = jnp.where(take, hist_here, hist_at)
            found = jnp.where(has, 1, found)
            acc = acc + jnp.sum(chunk)
            return (acc, found, dstar, sfx_at, hist_at)

        init = (_I32(0), _I32(0), _I32(0), _I32(0), _I32(0))
        _, _, dstar, sfx_at, hist_at = lax.fori_loop(0, 16, scan_body, init)
        prefix = (prefix << _U32(8)) | dstar.astype(_U32)
        count_above = count_above + (sfx_at - hist_at)
    return prefix, count_above


def _sc_select_body(cmax_hbm, maskrows_hbm,
                    widx_hbm, wval_hbm,
                    l1buf, hist, merged, tmp, candid, l2vals,
                    idxbuf, pos_st, pos2d, idx_st, val_st,
                    cnt_smem, shared):
    c = lax.axis_index("c")
    s = lax.axis_index("s")
    b = c * _WPR + s // _WPR
    q = s % _WPR
    lead = (s // _WPR) * _WPR
    lanes = _iota16()

    @pl.when(q == 0)
    def _():
        cnt_smem[0] = 0
        cnt_smem[1] = 0
    plsc.subcore_barrier()

    base = b * _NCHROW + q * _L1N
    with jax.named_scope("p1_load"):
        pltpu.sync_copy(cmax_hbm.at[pl.ds(base, _L1N)], l1buf)

    def key_body(g, _):
        v = l1buf[pl.ds(g * 16, 16)]
        l1buf[pl.ds(g * 16, 16)] = lax.bitcast_convert_type(_mono_key(v), jnp.float32)
        return 0
    lax.fori_loop(0, _L1N // 16, key_body, 0)

    def l1_keys(g):
        return lax.bitcast_convert_type(l1buf[pl.ds(g * 16, 16)], _U32)

    with jax.named_scope("p2_l1sel"):
        t1, _ = _select_kth(l1_keys, _L1N // 16, _K, hist, merged, tmp,
                            shared, s)

    def czero_body(i, _):
        candid[pl.ds(i * 16, 16)] = jnp.zeros((16,), _I32)
        return 0
    lax.fori_loop(0, _CANDCAP // 16, czero_body, 0)

    def coll_body(g, off):
        key = l1_keys(g)
        keep = jnp.logical_and(key >= t1, off < _CANDCAP - 16)
        p = q * _L1N + g * 16 + lanes
        ids = (p & (_CENC - 1)) * _TCH + (p >> 8)
        plsc.store_compressed(candid.at[pl.ds(off, 16)], ids, mask=keep)
        return off + _scalar(plsc.all_reduce_population_count(keep))
    with jax.named_scope("p3_collect"):
        mycount = lax.fori_loop(0, _L1N // 16, coll_body, _I32(0))

    rowbase = b * _NCHROW

    def gather_body(j, _):
        def gi_body(u, _2):
            idxbuf[pl.ds(u * 16, 16)] = (
                candid[pl.ds(j * 128 + u * 16, 16)] + rowbase)
            return 0
        lax.fori_loop(0, 8, gi_body, 0)
        pltpu.sync_copy(maskrows_hbm.at[idxbuf],
                        l2vals.at[pl.ds(j * 128, 128)])
        return 0
    ngather = (mycount + 127) // 128
    with jax.named_scope("p4_gather"):
        lax.fori_loop(0, ngather, gather_body, 0)

    def l2_keys(g):
        return _mono_key(l2vals[g >> 3, pl.ds((g & 7) * 16, 16)])

    n2 = mycount * (_CH // 16)
    with jax.named_scope("p5_l2sel"):
        t2, count_above2 = _select_kth(l2_keys, n2, _K, hist, merged, tmp,
                                       shared, s)
    need = _K - count_above2

    def cnt_body(g, carry):
        ns, nt = carry
        key = l2_keys(g)
        ns = ns + _scalar(plsc.all_reduce_population_count(key > t2))
        nt = nt + _scalar(plsc.all_reduce_population_count(key == t2))
        return (ns, nt)
    with jax.named_scope("p6_count"):
        ns_tot, nt_tot = lax.fori_loop(0, n2, cnt_body, (_I32(0), _I32(0)))
    sbase = plsc.fetch_and_add(cnt_smem.at[0], ns_tot, subcore_id=lead)
    tbase = plsc.fetch_and_add(cnt_smem.at[1], nt_tot, subcore_id=lead)

    def pfill_body(i, _):
        pos_st[pl.ds(i * 16, 16)] = _B * _K + lanes
        return 0
    lax.fori_loop(0, _K // 16, pfill_body, 0)

    out_base = b * _K

    def emit_body(g, carry):
        off, s_run, t_run = carry
        r = g >> 3
        col = (g & 7) * 16
        v = l2vals[r, pl.ds(col, 16)]
        key = _mono_key(v)
        strict = key > t2
        tie = key == t2
        id16 = plsc.load_gather(candid, [jnp.zeros((16,), _I32) + r])
        fidx = id16 * _CH + col + lanes
        cs = plsc.cumsum(jnp.where(strict, 1, 0).astype(_I32))
        ps = out_base + sbase + s_run + cs - 1
        ct = plsc.cumsum(jnp.where(tie, 1, 0).astype(_I32))
        ord_t = tbase + t_run + ct - 1
        tincl = jnp.logical_and(tie, ord_t < need)
        pt = out_base + count_above2 + ord_t
        plsc.store_compressed(pos_st.at[pl.ds(off, 16)], ps, mask=strict)
        plsc.store_compressed(idx_st.at[pl.ds(off, 16)], fidx, mask=strict)
        plsc.store_compressed(val_st.at[pl.ds(off, 16)], v, mask=strict)
        nso = _scalar(plsc.all_reduce_population_count(strict))
        off = off + nso
        plsc.store_compressed(pos_st.at[pl.ds(off, 16)], pt, mask=tincl)
        plsc.store_compressed(idx_st.at[pl.ds(off, 16)], fidx, mask=tincl)
        plsc.store_compressed(val_st.at[pl.ds(off, 16)], v, mask=tincl)
        nto = _scalar(plsc.all_reduce_population_count(tincl))
        return (off + nto, s_run + nso, t_run + _scalar(
            plsc.all_reduce_population_count(tie)))
    with jax.named_scope("p7_emit"):
        lax.fori_loop(0, n2, emit_body, (_I32(0), _I32(0), _I32(0)))

    def p2d_body(g, _):
        pos2d[g // 8, pl.ds((g % 8) * 16, 16)] = pos_st[pl.ds(g * 16, 16)]
        return 0
    lax.fori_loop(0, _K // 16, p2d_body, 0)
    with jax.named_scope("p8_scatter"):
      for j in range(_K // 128):
        pltpu.sync_copy(idx_st.at[pl.ds(j * 128, 128)],
                        widx_hbm.at[pos2d.at[j]])
        pltpu.sync_copy(val_st.at[pl.ds(j * 128, 128)],
                        wval_hbm.at[pos2d.at[j]])


def _sc_select(cmax_flat, maskrows):
    mesh = plsc.VectorSubcoreMesh(core_axis_name="c", subcore_axis_name="s")
    f = pl.kernel(
        _sc_select_body,
        out_type=(
            jax.ShapeDtypeStruct((_B * _K + _PAD,), jnp.int32),
            jax.ShapeDtypeStruct((_B * _K + _PAD,), jnp.float32),
        ),
        mesh=mesh,
        scratch_types=[
            pltpu.VMEM((_L1N,), jnp.float32),
            pltpu.VMEM((4096,), jnp.int32),
            pltpu.VMEM((256,), jnp.int32),
            pltpu.VMEM((256,), jnp.int32),
            pltpu.VMEM((_CANDCAP,), jnp.int32),
            pltpu.VMEM((_CANDCAP, _CH), jnp.float32),
            pltpu.VMEM((128,), jnp.int32),
            pltpu.VMEM((_K,), jnp.int32),
            pltpu.VMEM((_K // 128, 128), jnp.int32),
            pltpu.VMEM((_K,), jnp.int32),
            pltpu.VMEM((_K,), jnp.float32),
            pltpu.SMEM((8,), jnp.int32),
            pltpu.VMEM_SHARED((16, 256), jnp.int32),
        ],
        compiler_params=pltpu.CompilerParams(needs_layout_passes=False),
    )
    return f(cmax_flat, maskrows)


def kernel(x, W_up, b_up, W_mask, b_mask):
    mask, zeros, cmax, m_row, z_row = _dense_pass(x, W_mask, b_mask)
    M = m_row[:, 0, 0]
    Z = z_row[:, 0, 0]

    cmax_flat = cmax.reshape(_B * _NCHROW)
    maskrows = mask.reshape(_B * _NCHROW, _CH)
    widx_raw, wval_raw = _sc_select(cmax_flat, maskrows)
    win_idx = widx_raw[:_B * _K].reshape(_B, _K)
    win_vals = wval_raw[:_B * _K].reshape(_B, _K)

    sm_vals = jnp.exp(win_vals - M[:, None]) / Z[:, None]

    o_idx = win_idx // _T
    t_idx = win_idx % _T
    x_cols = jnp.take_along_axis(
        x, t_idx[:, None, :], axis=2)
    w_rows = W_up[o_idx]
    up_vals = jnp.einsum('bkc,bck->bk', w_rows, x_cols) + b_up[o_idx]

    out_flat = zeros.reshape(_B, _FLAT)
    rows = jnp.arange(_B)[:, None]
    out_flat = out_flat.at[rows, win_idx].set(sm_vals * up_vals)
    return out_flat.reshape(_B, _CENC, _T)


# speedup vs baseline: 1.9573x; 1.9573x over previous
import functools

import jax
import jax.numpy as jnp
from jax import lax
from jax.experimental import pallas as pl
from jax.experimental.pallas import tpu as pltpu
from jax.experimental.pallas import tpu_sc as plsc

_B, _CIN, _CENC, _T = 8, 128, 256, 32768
_FLAT = _CENC * _T
_K = 512
_CH = 128
_NCHROW = _FLAT // _CH
_TCH = _T // _CH
_TBLK = 8192
_NT = _T // _TBLK

_WPR = 4
_L1N = _NCHROW // _WPR
_CANDCAP = 640
_PAD = 128


def _dense_pass_body(x_ref, wm_ref, bm_ref,
                     mask_ref, zeros_ref, cmax_ref, m_out, z_out,
                     m_run, s_run):
    t = pl.program_id(1)
    xt = x_ref[0]
    mask = jnp.dot(wm_ref[...], xt, preferred_element_type=jnp.float32)
    mask = mask + bm_ref[...]
    mask_ref[0] = mask
    zeros_ref[0] = jnp.zeros_like(mask)
    cm = jnp.max(mask.reshape(_CENC, _TBLK // _CH, _CH), axis=-1)
    cmax_ref[0] = cm.T
    tm = jnp.max(cm)
    ts = jnp.sum(jnp.exp(mask - tm))

    @pl.when(t == 0)
    def _():
        m_run[0] = tm
        s_run[0] = ts

    @pl.when(t > 0)
    def _():
        m_old = m_run[0]
        s_old = s_run[0]
        m_new = jnp.maximum(m_old, tm)
        s_run[0] = s_old * jnp.exp(m_old - m_new) + ts * jnp.exp(tm - m_new)
        m_run[0] = m_new

    @pl.when(t == _NT - 1)
    def _():
        m_out[0, 0, :] = jnp.full((128,), m_run[0], jnp.float32)
        z_out[0, 0, :] = jnp.full((128,), s_run[0], jnp.float32)


def _dense_pass(x, W_mask, b_mask):
    grid = (_B, _NT)
    out_shapes = (
        jax.ShapeDtypeStruct((_B, _CENC, _T), jnp.float32),
        jax.ShapeDtypeStruct((_B, _CENC, _T), jnp.float32),
        jax.ShapeDtypeStruct((_B, _TCH, _CENC), jnp.float32),
        jax.ShapeDtypeStruct((_B, 1, 128), jnp.float32),
        jax.ShapeDtypeStruct((_B, 1, 128), jnp.float32),
    )
    return pl.pallas_call(
        _dense_pass_body,
        grid=grid,
        in_specs=[
            pl.BlockSpec((1, _CIN, _TBLK), lambda b, t: (b, 0, t)),
            pl.BlockSpec((_CENC, _CIN), lambda b, t: (0, 0)),
            pl.BlockSpec((_CENC, 1), lambda b, t: (0, 0)),
        ],
        out_specs=(
            pl.BlockSpec((1, _CENC, _TBLK), lambda b, t: (b, 0, t)),
            pl.BlockSpec((1, _CENC, _TBLK), lambda b, t: (b, 0, t)),
            pl.BlockSpec((1, _TBLK // _CH, _CENC), lambda b, t: (b, t, 0)),
            pl.BlockSpec((1, 1, 128), lambda b, t: (b, 0, 0)),
            pl.BlockSpec((1, 1, 128), lambda b, t: (b, 0, 0)),
        ),
        out_shape=out_shapes,
        scratch_shapes=[
            pltpu.SMEM((1,), jnp.float32),
            pltpu.SMEM((1,), jnp.float32),
        ],
        compiler_params=pltpu.CompilerParams(
            dimension_semantics=("arbitrary", "arbitrary"),
        ),
    )(x, W_mask, b_mask.reshape(_CENC, 1))


_U32 = jnp.uint32
_I32 = jnp.int32


def _iota16():
    return jax.lax.iota(_I32, 16)


def _mono_key(v):
    kv = lax.bitcast_convert_type(v, _U32)
    sgn = kv >> _U32(31)
    m = jnp.where(sgn == _U32(1), _U32(0xFFFFFFFF), _U32(0x80000000))
    return kv ^ m


def _scalar(x16):
    return jnp.max(x16)


def _select_kth(get_keys, n_groups, k, hist, merged, tmp, shared, s):
    prefix = _U32(0)
    count_above = _I32(0)
    gbase = (s // 4) * 4
    for r in range(4):
        shift = 24 - 8 * r

        def zero_body(i, _):
            hist[pl.ds(i * 16, 16)] = jnp.zeros((16,), _I32)
            return 0
        lax.fori_loop(0, 256, zero_body, 0)

        pfx = prefix
        lanes = _iota16()

        def fill_body(g, _):
            key = get_keys(g)
            if r == 0:
                act = jnp.ones((16,), jnp.bool_)
            else:
                act = (key >> _U32(shift + 8)) == pfx
            digit = ((key >> _U32(shift)) & _U32(0xFF)).astype(_I32)
            addr = lanes * 256 + digit
            plsc.addupdate_scatter(hist, [addr],
                                   jnp.where(act, 1, 0).astype(_I32))
            return 0
        lax.fori_loop(0, n_groups, fill_body, 0)

        def merge_body(gg, _):
            acc = hist[pl.ds(gg * 16, 16)]
            for l in range(1, 16):
                acc = acc + hist[pl.ds(l * 256 + gg * 16, 16)]
            merged[pl.ds(gg * 16, 16)] = acc
            return 0
        lax.fori_loop(0, 16, merge_body, 0)

        pltpu.sync_copy(merged, shared.at[s])
        plsc.subcore_barrier()
        for j in range(_WPR):
            pltpu.sync_copy(shared.at[gbase + j], tmp)

            def add_body(gg, _):
                if j == 0:
                    merged[pl.ds(gg * 16, 16)] = tmp[pl.ds(gg * 16, 16)]
                else:
                    merged[pl.ds(gg * 16, 16)] = (
                        merged[pl.ds(gg * 16, 16)] + tmp[pl.ds(gg * 16, 16)])
                return 0
            lax.fori_loop(0, 16, add_body, 0)
        plsc.subcore_barrier()

        ca = count_above

        def scan_body(i, carry):
            acc, found, dstar, sfx_at, hist_at = carry
            g = 15 - i
            chunk = merged[pl.ds(g * 16, 16)]
            sfx = lax.rev(plsc.cumsum(lax.rev(chunk, (0,))), (0,))
            tot = sfx + acc
            cond = (ca + tot) >= k
            npos = _scalar(plsc.all_reduce_population_count(cond))
            has = npos > 0
            condi = jnp.where(cond, 1, 0).astype(_I32)
            ffs_rev = _scalar(plsc.all_reduce_ffs(lax.rev(condi, (0,)) == 1))
            dl = 15 - ffs_rev
            sel = lanes == dl
            z16 = jnp.zeros((16,), _I32)
            sfx_here = _scalar(jnp.where(sel, tot, z16))
            hist_here = _scalar(jnp.where(sel, chunk, z16))
            take = jnp.logical_and(has, found == 0)
            dstar = jnp.where(take, g * 16 + dl, dstar)
            sfx_at = jnp.where(take, sfx_here, sfx_at)
            hist_at = jnp.where(take, hist_here, hist_at)
            found = jnp.where(has, 1, found)
            acc = acc + jnp.sum(chunk)
            return (acc, found, dstar, sfx_at, hist_at)

        init = (_I32(0), _I32(0), _I32(0), _I32(0), _I32(0))
        _, _, dstar, sfx_at, hist_at = lax.fori_loop(0, 16, scan_body, init)
        prefix = (prefix << _U32(8)) | dstar.astype(_U32)
        count_above = count_above + (sfx_at - hist_at)
    return prefix, count_above


def _sc_select_body(cmax_hbm, maskrows_hbm,
                    widx_hbm, wval_hbm,
                    l1buf, hist, merged, tmp, candid, l2vals,
                    idxbuf, pos_st, pos2d, idx_st, val_st,
                    cnt_smem, shared):
    c = lax.axis_index("c")
    s = lax.axis_index("s")
    b = c * _WPR + s // _WPR
    q = s % _WPR
    lead = (s // _WPR) * _WPR
    lanes = _iota16()

    @pl.when(q == 0)
    def _():
        cnt_smem[0] = 0
        cnt_smem[1] = 0
    plsc.subcore_barrier()

    base = b * _NCHROW + q * _L1N
    with jax.named_scope("p1_load"):
        pltpu.sync_copy(cmax_hbm.at[pl.ds(base, _L1N)], l1buf)

    def key_body(g, _):
        v = l1buf[pl.ds(g * 16, 16)]
        l1buf[pl.ds(g * 16, 16)] = lax.bitcast_convert_type(_mono_key(v), jnp.float32)
        return 0
    lax.fori_loop(0, _L1N // 16, key_body, 0)

    def l1_keys(g):
        return lax.bitcast_convert_type(l1buf[pl.ds(g * 16, 16)], _U32)

    with jax.named_scope("p2_l1sel"):
        t1, _ = _select_kth(l1_keys, _L1N // 16, _K, hist, merged, tmp,
                            shared, s)

    def czero_body(i, _):
        candid[pl.ds(i * 16, 16)] = jnp.zeros((16,), _I32)
        return 0
    lax.fori_loop(0, _CANDCAP // 16, czero_body, 0)

    def coll_body(g, off):
        key = l1_keys(g)
        keep = jnp.logical_and(key >= t1, off < _CANDCAP - 16)
        p = q * _L1N + g * 16 + lanes
        ids = (p & (_CENC - 1)) * _TCH + (p >> 8)
        plsc.store_compressed(candid.at[pl.ds(off, 16)], ids, mask=keep)
        return off + _scalar(plsc.all_reduce_population_count(keep))
    with jax.named_scope("p3_collect"):
        mycount = lax.fori_loop(0, _L1N // 16, coll_body, _I32(0))

    rowbase = b * _NCHROW

    def gather_body(j, _):
        def gi_body(u, _2):
            idxbuf[pl.ds(u * 16, 16)] = (
                candid[pl.ds(j * 128 + u * 16, 16)] + rowbase)
            return 0
        lax.fori_loop(0, 8, gi_body, 0)
        pltpu.sync_copy(maskrows_hbm.at[idxbuf],
                        l2vals.at[pl.ds(j * 128, 128)])
        return 0
    ngather = (mycount + 127) // 128
    with jax.named_scope("p4_gather"):
        lax.fori_loop(0, ngather, gather_body, 0)

    def l2_keys(g):
        return _mono_key(l2vals[g >> 3, pl.ds((g & 7) * 16, 16)])

    n2 = mycount * (_CH // 16)
    with jax.named_scope("p5_l2sel"):
        t2, count_above2 = _select_kth(l2_keys, n2, _K, hist, merged, tmp,
                                       shared, s)
    need = _K - count_above2

    def cnt_body(g, carry):
        ns, nt = carry
        key = l2_keys(g)
        ns = ns + _scalar(plsc.all_reduce_population_count(key > t2))
        nt = nt + _scalar(plsc.all_reduce_population_count(key == t2))
        return (ns, nt)
    with jax.named_scope("p6_count"):
        ns_tot, nt_tot = lax.fori_loop(0, n2, cnt_body, (_I32(0), _I32(0)))
    sbase = plsc.fetch_and_add(cnt_smem.at[0], ns_tot, subcore_id=lead)
    tbase = plsc.fetch_and_add(cnt_smem.at[1], nt_tot, subcore_id=lead)

    wid = c * 16 + s
    def pfill_body(i, _):
        pos_st[pl.ds(i * 16, 16)] = _B * _K + wid * 16 + lanes
        return 0
    lax.fori_loop(0, _K // 16, pfill_body, 0)

    out_base = b * _K

    def emit_body(g, carry):
        off, s_run, t_run = carry
        r = g >> 3
        col = (g & 7) * 16
        v = l2vals[r, pl.ds(col, 16)]
        key = _mono_key(v)
        strict = key > t2
        tie = key == t2
        id16 = plsc.load_gather(candid, [jnp.zeros((16,), _I32) + r])
        fidx = id16 * _CH + col + lanes
        cs = plsc.cumsum(jnp.where(strict, 1, 0).astype(_I32))
        ps = out_base + sbase + s_run + cs - 1
        ct = plsc.cumsum(jnp.where(tie, 1, 0).astype(_I32))
        ord_t = tbase + t_run + ct - 1
        tincl = jnp.logical_and(tie, ord_t < need)
        pt = out_base + count_above2 + ord_t
        plsc.store_compressed(pos_st.at[pl.ds(off, 16)], ps, mask=strict)
        plsc.store_compressed(idx_st.at[pl.ds(off, 16)], fidx, mask=strict)
        plsc.store_compressed(val_st.at[pl.ds(off, 16)], v, mask=strict)
        nso = _scalar(plsc.all_reduce_population_count(strict))
        off = off + nso
        plsc.store_compressed(pos_st.at[pl.ds(off, 16)], pt, mask=tincl)
        plsc.store_compressed(idx_st.at[pl.ds(off, 16)], fidx, mask=tincl)
        plsc.store_compressed(val_st.at[pl.ds(off, 16)], v, mask=tincl)
        nto = _scalar(plsc.all_reduce_population_count(tincl))
        return (off + nto, s_run + nso, t_run + _scalar(
            plsc.all_reduce_population_count(tie)))
    with jax.named_scope("p7_emit"):
        nemit, _, _ = lax.fori_loop(0, n2, emit_body,
                                    (_I32(0), _I32(0), _I32(0)))

    def p2d_body(g, _):
        pos2d[g // 8, pl.ds((g % 8) * 16, 16)] = pos_st[pl.ds(g * 16, 16)]
        return 0
    lax.fori_loop(0, _K // 16, p2d_body, 0)

    with jax.named_scope("p8_scatter"):
        def sc_body(j, _):
            pltpu.sync_copy(idx_st.at[pl.ds(j * 128, 128)],
                            widx_hbm.at[pos2d.at[j]])
            pltpu.sync_copy(val_st.at[pl.ds(j * 128, 128)],
                            wval_hbm.at[pos2d.at[j]])
            return 0
        lax.fori_loop(0, (nemit + 127) // 128, sc_body, 0)


def _sc_select(cmax_flat, maskrows):
    mesh = plsc.VectorSubcoreMesh(core_axis_name="c", subcore_axis_name="s")
    f = pl.kernel(
        _sc_select_body,
        out_type=(
            jax.ShapeDtypeStruct((_B * _K + _PAD,), jnp.int32),
            jax.ShapeDtypeStruct((_B * _K + _PAD,), jnp.float32),
        ),
        mesh=mesh,
        scratch_types=[
            pltpu.VMEM((_L1N,), jnp.float32),
            pltpu.VMEM((4096,), jnp.int32),
            pltpu.VMEM((256,), jnp.int32),
            pltpu.VMEM((256,), jnp.int32),
            pltpu.VMEM((_CANDCAP,), jnp.int32),
            pltpu.VMEM((_CANDCAP, _CH), jnp.float32),
            pltpu.VMEM((128,), jnp.int32),
            pltpu.VMEM((_K,), jnp.int32),
            pltpu.VMEM((_K // 128, 128), jnp.int32),
            pltpu.VMEM((_K,), jnp.int32),
            pltpu.VMEM((_K,), jnp.float32),
            pltpu.SMEM((8,), jnp.int32),
            pltpu.VMEM_SHARED((16, 256), jnp.int32),
        ],
        compiler_params=pltpu.CompilerParams(needs_layout_passes=False),
    )
    return f(cmax_flat, maskrows)


def kernel(x, W_up, b_up, W_mask, b_mask):
    mask, zeros, cmax, m_row, z_row = _dense_pass(x, W_mask, b_mask)
    M = m_row[:, 0, 0]
    Z = z_row[:, 0, 0]

    cmax_flat = cmax.reshape(_B * _NCHROW)
    maskrows = mask.reshape(_B * _NCHROW, _CH)
    widx_raw, wval_raw = _sc_select(cmax_flat, maskrows)
    win_idx = widx_raw[:_B * _K].reshape(_B, _K)
    win_vals = wval_raw[:_B * _K].reshape(_B, _K)

    sm_vals = jnp.exp(win_vals - M[:, None]) / Z[:, None]

    o_idx = win_idx // _T
    t_idx = win_idx % _T
    x_cols = jnp.take_along_axis(
        x, t_idx[:, None, :], axis=2)
    w_rows = W_up[o_idx]
    up_vals = jnp.einsum('bkc,bck->bk', w_rows, x_cols) + b_up[o_idx]

    out_flat = zeros.reshape(_B, _FLAT)
    rows = jnp.arange(_B)[:, None]
    out_flat = out_flat.at[rows, win_idx].set(sm_vals * up_vals)
    return out_flat.reshape(_B, _CENC, _T)


# speedup vs baseline: 1.9603x; 1.0015x over previous
import functools

import jax
import jax.numpy as jnp
from jax import lax
from jax.experimental import pallas as pl
from jax.experimental.pallas import tpu as pltpu
from jax.experimental.pallas import tpu_sc as plsc

_B, _CIN, _CENC, _T = 8, 128, 256, 32768
_FLAT = _CENC * _T
_K = 512
_CH = 128
_NCHROW = _FLAT // _CH
_TCH = _T // _CH
_TBLK = 8192
_NT = _T // _TBLK

_WPR = 4
_L1N = _NCHROW // _WPR
_CANDCAP = 640
_PAD = 128


def _dense_pass_body(x_ref, wm_ref, bm_ref,
                     mask_ref, zeros_ref, cmax_ref, m_out, z_out,
                     m_run, s_run):
    t = pl.program_id(1)
    xt = x_ref[0]
    mask = jnp.dot(wm_ref[...], xt, preferred_element_type=jnp.float32)
    mask = mask + bm_ref[...]
    mask_ref[0] = mask
    zeros_ref[0] = jnp.zeros_like(mask)
    cm = jnp.max(mask.reshape(_CENC, _TBLK // _CH, _CH), axis=-1)
    cmax_ref[0] = cm.T
    tm = jnp.max(cm)
    ts = jnp.sum(jnp.exp(mask - tm))

    @pl.when(t == 0)
    def _():
        m_run[0] = tm
        s_run[0] = ts

    @pl.when(t > 0)
    def _():
        m_old = m_run[0]
        s_old = s_run[0]
        m_new = jnp.maximum(m_old, tm)
        s_run[0] = s_old * jnp.exp(m_old - m_new) + ts * jnp.exp(tm - m_new)
        m_run[0] = m_new

    @pl.when(t == _NT - 1)
    def _():
        m_out[0, 0, :] = jnp.full((128,), m_run[0], jnp.float32)
        z_out[0, 0, :] = jnp.full((128,), s_run[0], jnp.float32)


def _dense_pass(x, W_mask, b_mask):
    grid = (_B, _NT)
    out_shapes = (
        jax.ShapeDtypeStruct((_B, _CENC, _T), jnp.float32),
        jax.ShapeDtypeStruct((_B, _CENC, _T), jnp.float32),
        jax.ShapeDtypeStruct((_B, _TCH, _CENC), jnp.float32),
        jax.ShapeDtypeStruct((_B, 1, 128), jnp.float32),
        jax.ShapeDtypeStruct((_B, 1, 128), jnp.float32),
    )
    return pl.pallas_call(
        _dense_pass_body,
        grid=grid,
        in_specs=[
            pl.BlockSpec((1, _CIN, _TBLK), lambda b, t: (b, 0, t)),
            pl.BlockSpec((_CENC, _CIN), lambda b, t: (0, 0)),
            pl.BlockSpec((_CENC, 1), lambda b, t: (0, 0)),
        ],
        out_specs=(
            pl.BlockSpec((1, _CENC, _TBLK), lambda b, t: (b, 0, t)),
            pl.BlockSpec((1, _CENC, _TBLK), lambda b, t: (b, 0, t)),
            pl.BlockSpec((1, _TBLK // _CH, _CENC), lambda b, t: (b, t, 0)),
            pl.BlockSpec((1, 1, 128), lambda b, t: (b, 0, 0)),
            pl.BlockSpec((1, 1, 128), lambda b, t: (b, 0, 0)),
        ),
        out_shape=out_shapes,
        scratch_shapes=[
            pltpu.SMEM((1,), jnp.float32),
            pltpu.SMEM((1,), jnp.float32),
        ],
        compiler_params=pltpu.CompilerParams(
            dimension_semantics=("arbitrary", "arbitrary"),
        ),
    )(x, W_mask, b_mask.reshape(_CENC, 1))


_U32 = jnp.uint32
_I32 = jnp.int32


def _iota16():
    return jax.lax.iota(_I32, 16)


def _mono_key(v):
    kv = lax.bitcast_convert_type(v, _U32)
    sgn = kv >> _U32(31)
    m = jnp.where(sgn == _U32(1), _U32(0xFFFFFFFF), _U32(0x80000000))
    return kv ^ m


def _scalar(x16):
    return jnp.max(x16)


def _select_kth(get_keys, n_groups, k, hist, merged, tmp, shared, s):
    prefix = _U32(0)
    count_above = _I32(0)
    gbase = (s // 4) * 4
    for r in range(4):
        shift = 24 - 8 * r

        def zero_body(i, _):
            hist[pl.ds(i * 16, 16)] = jnp.zeros((16,), _I32)
            return 0
        lax.fori_loop(0, 256, zero_body, 0)

        pfx = prefix
        lanes = _iota16()

        def fill_body(g, _):
            key = get_keys(g)
            if r == 0:
                act = jnp.ones((16,), jnp.bool_)
            else:
                act = (key >> _U32(shift + 8)) == pfx
            digit = ((key >> _U32(shift)) & _U32(0xFF)).astype(_I32)
            addr = lanes * 256 + digit
            plsc.addupdate_scatter(hist, [addr],
                                   jnp.where(act, 1, 0).astype(_I32))
            return 0
        lax.fori_loop(0, n_groups, fill_body, 0)

        def merge_body(gg, _):
            acc = hist[pl.ds(gg * 16, 16)]
            for l in range(1, 16):
                acc = acc + hist[pl.ds(l * 256 + gg * 16, 16)]
            merged[pl.ds(gg * 16, 16)] = acc
            return 0
        lax.fori_loop(0, 16, merge_body, 0)

        pltpu.sync_copy(merged, shared.at[s])
        plsc.subcore_barrier()
        for j in range(_WPR):
            pltpu.sync_copy(shared.at[gbase + j], tmp)

            def add_body(gg, _):
                if j == 0:
                    merged[pl.ds(gg * 16, 16)] = tmp[pl.ds(gg * 16, 16)]
                else:
                    merged[pl.ds(gg * 16, 16)] = (
                        merged[pl.ds(gg * 16, 16)] + tmp[pl.ds(gg * 16, 16)])
                return 0
            lax.fori_loop(0, 16, add_body, 0)
        plsc.subcore_barrier()

        ca = count_above

        def scan_body(i, carry):
            acc, found, dstar, sfx_at, hist_at = carry
            g = 15 - i
            chunk = merged[pl.ds(g * 16, 16)]
            sfx = lax.rev(plsc.cumsum(lax.rev(chunk, (0,))), (0,))
            tot = sfx + acc
            cond = (ca + tot) >= k
            npos = _scalar(plsc.all_reduce_population_count(cond))
            has = npos > 0
            condi = jnp.where(cond, 1, 0).astype(_I32)
            ffs_rev = _scalar(plsc.all_reduce_ffs(lax.rev(condi, (0,)) == 1))
            dl = 15 - ffs_rev
            sel = lanes == dl
            z16 = jnp.zeros((16,), _I32)
            sfx_here = _scalar(jnp.where(sel, tot, z16))
            hist_here = _scalar(jnp.where(sel, chunk, z16))
            take = jnp.logical_and(has, found == 0)
            dstar = jnp.where(take, g * 16 + dl, dstar)
            sfx_at = jnp.where(take, sfx_here, sfx_at)
            hist_at = jnp.where(take, hist_here, hist_at)
            found = jnp.where(has, 1, found)
            acc = acc + jnp.sum(chunk)
            return (acc, found, dstar, sfx_at, hist_at)

        init = (_I32(0), _I32(0), _I32(0), _I32(0), _I32(0))
        _, _, dstar, sfx_at, hist_at = lax.fori_loop(0, 16, scan_body, init)
        prefix = (prefix << _U32(8)) | dstar.astype(_U32)
        count_above = count_above + (sfx_at - hist_at)
    return prefix, count_above


def _sc_select_body(cmax_hbm, maskrows_hbm,
                    widx_hbm, wval_hbm,
                    l1buf, hist, merged, tmp, candid, l2vals,
                    idxbuf, pos_st, pos2d, idx_st, val_st,
                    cnt_smem, shared):
    c = lax.axis_index("c")
    s = lax.axis_index("s")
    b = c * _WPR + s // _WPR
    q = s % _WPR
    lead = (s // _WPR) * _WPR
    lanes = _iota16()

    @pl.when(q == 0)
    def _():
        cnt_smem[0] = 0
        cnt_smem[1] = 0
    plsc.subcore_barrier()

    base = b * _NCHROW + q * _L1N
    with jax.named_scope("p1_load"):
        pltpu.sync_copy(cmax_hbm.at[pl.ds(base, _L1N)], l1buf)

    def key_body(g, _):
        v = l1buf[pl.ds(g * 16, 16)]
        l1buf[pl.ds(g * 16, 16)] = lax.bitcast_convert_type(_mono_key(v), jnp.float32)
        return 0
    lax.fori_loop(0, _L1N // 16, key_body, 0)

    def l1_keys(g):
        return lax.bitcast_convert_type(l1buf[pl.ds(g * 16, 16)], _U32)

    with jax.named_scope("p2_l1sel"):
        t1, _ = _select_kth(l1_keys, _L1N // 16, _K, hist, merged, tmp,
                            shared, s)

    def czero_body(i, _):
        candid[pl.ds(i * 16, 16)] = jnp.zeros((16,), _I32)
        return 0
    lax.fori_loop(0, _CANDCAP // 16, czero_body, 0)

    def coll_body(g, off):
        key = l1_keys(g)
        keep = jnp.logical_and(key >= t1, off < _CANDCAP - 16)
        p = q * _L1N + g * 16 + lanes
        ids = (p & (_CENC - 1)) * _TCH + (p >> 8)
        plsc.store_compressed(candid.at[pl.ds(off, 16)], ids, mask=keep)
        return off + _scalar(plsc.all_reduce_population_count(keep))
    with jax.named_scope("p3_collect"):
        mycount = lax.fori_loop(0, _L1N // 16, coll_body, _I32(0))

    rowbase = b * _NCHROW

    def gather_body(j, _):
        def gi_body(u, _2):
            idxbuf[pl.ds(u * 16, 16)] = (
                candid[pl.ds(j * 128 + u * 16, 16)] + rowbase)
            return 0
        lax.fori_loop(0, 8, gi_body, 0)
        pltpu.sync_copy(maskrows_hbm.at[idxbuf],
                        l2vals.at[pl.ds(j * 128, 128)])
        return 0
    ngather = (mycount + 127) // 128
    with jax.named_scope("p4_gather"):
        lax.fori_loop(0, ngather, gather_body, 0)

    def l2_keys(g):
        return _mono_key(l2vals[g >> 3, pl.ds((g & 7) * 16, 16)])

    n2 = mycount * (_CH // 16)
    with jax.named_scope("p5_l2sel"):
        t2, count_above2 = _select_kth(l2_keys, n2, _K, hist, merged, tmp,
                                       shared, s)
    need = _K - count_above2

    def cnt_body(g, carry):
        ns, nt = carry
        key = l2_keys(g)
        ns = ns + _scalar(plsc.all_reduce_population_count(key > t2))
        nt = nt + _scalar(plsc.all_reduce_population_count(key == t2))
        return (ns, nt)
    with jax.named_scope("p6_count"):
        ns_tot, nt_tot = lax.fori_loop(0, n2, cnt_body, (_I32(0), _I32(0)))
    sbase = plsc.fetch_and_add(cnt_smem.at[0], ns_tot, subcore_id=lead)
    tbase = plsc.fetch_and_add(cnt_smem.at[1], nt_tot, subcore_id=lead)

    wid = c * 16 + s
    def pfill_body(i, _):
        pos_st[pl.ds(i * 16, 16)] = _B * _K + wid * 16 + lanes
        return 0
    lax.fori_loop(0, _K // 16, pfill_body, 0)

    out_base = b * _K

    def emit_body(g, carry):
        off, s_run, t_run = carry
        r = g >> 3
        col = (g & 7) * 16
        v = l2vals[r, pl.ds(col, 16)]
        key = _mono_key(v)
        strict = key > t2
        tie = key == t2
        id16 = plsc.load_gather(candid, [jnp.zeros((16,), _I32) + r])
        fidx = id16 * _CH + col + lanes
        cs = plsc.cumsum(jnp.where(strict, 1, 0).astype(_I32))
        ps = out_base + sbase + s_run + cs - 1
        ct = plsc.cumsum(jnp.where(tie, 1, 0).astype(_I32))
        ord_t = tbase + t_run + ct - 1
        tincl = jnp.logical_and(tie, ord_t < need)
        pt = out_base + count_above2 + ord_t
        plsc.store_compressed(pos_st.at[pl.ds(off, 16)], ps, mask=strict)
        plsc.store_compressed(idx_st.at[pl.ds(off, 16)], fidx, mask=strict)
        plsc.store_compressed(val_st.at[pl.ds(off, 16)], v, mask=strict)
        nso = _scalar(plsc.all_reduce_population_count(strict))
        off = off + nso
        plsc.store_compressed(pos_st.at[pl.ds(off, 16)], pt, mask=tincl)
        plsc.store_compressed(idx_st.at[pl.ds(off, 16)], fidx, mask=tincl)
        plsc.store_compressed(val_st.at[pl.ds(off, 16)], v, mask=tincl)
        nto = _scalar(plsc.all_reduce_population_count(tincl))
        return (off + nto, s_run + nso, t_run + _scalar(
            plsc.all_reduce_population_count(tie)))
    with jax.named_scope("p7_emit"):
        nemit, _, _ = lax.fori_loop(0, n2, emit_body,
                                    (_I32(0), _I32(0), _I32(0)))

    def p2d_body(g, _):
        pos2d[g // 8, pl.ds((g % 8) * 16, 16)] = pos_st[pl.ds(g * 16, 16)]
        return 0
    lax.fori_loop(0, _K // 16, p2d_body, 0)

    with jax.named_scope("p8_scatter"):
        def sc_body(j, _):
            pltpu.sync_copy(idx_st.at[pl.ds(j * 128, 128)],
                            widx_hbm.at[pos2d.at[j]])
            pltpu.sync_copy(val_st.at[pl.ds(j * 128, 128)],
                            wval_hbm.at[pos2d.at[j]])
            return 0
        lax.fori_loop(0, (nemit + 127) // 128, sc_body, 0)


def _sc_select(cmax_flat, maskrows):
    mesh = plsc.VectorSubcoreMesh(core_axis_name="c", subcore_axis_name="s")
    f = pl.kernel(
        _sc_select_body,
        out_type=(
            jax.ShapeDtypeStruct((_B * _K + _PAD,), jnp.int32),
            jax.ShapeDtypeStruct((_B * _K + _PAD,), jnp.float32),
        ),
        mesh=mesh,
        scratch_types=[
            pltpu.VMEM((_L1N,), jnp.float32),
            pltpu.VMEM((4096,), jnp.int32),
            pltpu.VMEM((256,), jnp.int32),
            pltpu.VMEM((256,), jnp.int32),
            pltpu.VMEM((_CANDCAP,), jnp.int32),
            pltpu.VMEM((_CANDCAP, _CH), jnp.float32),
            pltpu.VMEM((128,), jnp.int32),
            pltpu.VMEM((_K,), jnp.int32),
            pltpu.VMEM((_K // 128, 128), jnp.int32),
            pltpu.VMEM((_K,), jnp.int32),
            pltpu.VMEM((_K,), jnp.float32),
            pltpu.SMEM((8,), jnp.int32),
            pltpu.VMEM_SHARED((16, 256), jnp.int32),
        ],
        compiler_params=pltpu.CompilerParams(needs_layout_passes=False),
    )
    return f(cmax_flat, maskrows)


def kernel(x, W_up, b_up, W_mask, b_mask):
    mask, zeros, cmax, m_row, z_row = _dense_pass(x, W_mask, b_mask)
    M = m_row[:, 0, 0]
    Z = z_row[:, 0, 0]

    cmax_flat = cmax.reshape(_B * _NCHROW)
    maskrows = mask.reshape(_B * _NCHROW, _CH)
    widx_raw, wval_raw = _sc_select(cmax_flat, maskrows)
    win_idx = widx_raw[:_B * _K].reshape(_B, _K)
    win_vals = wval_raw[:_B * _K].reshape(_B, _K)

    sm_vals = jnp.exp(win_vals - M[:, None]) / Z[:, None]

    o_idx = win_idx // _T
    t_idx = win_idx % _T
    x_cols = jnp.take_along_axis(
        x, t_idx[:, None, :], axis=2)
    w_rows = W_up[o_idx]
    up_vals = jnp.einsum('bkc,bck->bk', w_rows, x_cols) + b_up[o_idx]

    rows = jnp.arange(_B)[:, None]
    out = zeros.at[rows, o_idx, t_idx].set(
        sm_vals * up_vals, unique_indices=True, mode="promise_in_bounds")
    return out
